# baseline (speedup 1.0000x reference)
import functools

import jax
import jax.numpy as jnp
from jax import lax
from jax.experimental import pallas as pl
from jax.experimental.pallas import tpu as pltpu
from jax.experimental.pallas import tpu_sc as plsc

NC = 2
NS = 16
NW = NC * NS
LANES = 16
CHUNK = 128
KP0_NUM, KP1_NUM = 2, 3


def _sc_mesh():
  return plsc.VectorSubcoreMesh(core_axis_name="c", subcore_axis_name="s")


def _deg_pass(col3, ew3, n):
  kch = col3.shape[1]

  @functools.partial(
      pl.kernel,
      out_type=jax.ShapeDtypeStruct((NW * n,), jnp.float32),
      mesh=_sc_mesh(),
      scratch_types=[
          pltpu.VMEM((kch, CHUNK), jnp.int32),
          pltpu.VMEM((kch, CHUNK), jnp.float32),
          pltpu.VMEM((n,), jnp.float32),
      ],
      compiler_params=pltpu.CompilerParams(needs_layout_passes=False),
  )
  def k(col_hbm, ew_hbm, out_hbm, col_v, ew_v, deg_v):
    cid = lax.axis_index("c")
    sid = lax.axis_index("s")
    wid = sid * NC + cid
    pltpu.sync_copy(col_hbm.at[wid], col_v)
    pltpu.sync_copy(ew_hbm.at[wid], ew_v)

    zero = jnp.zeros((LANES,), jnp.float32)

    def zbody(i, carry):
      deg_v[pl.ds(i * LANES, LANES)] = zero
      return carry

    lax.fori_loop(0, n // LANES, zbody, 0)

    def cbody(j, carry):
      for g in range(CHUNK // LANES):
        idx = col_v[j, pl.ds(g * LANES, LANES)]
        val = ew_v[j, pl.ds(g * LANES, LANES)]
        plsc.addupdate_scatter(deg_v, [idx], val)
      return carry

    lax.fori_loop(0, kch, cbody, 0)
    pltpu.sync_copy(deg_v, out_hbm.at[pl.ds(wid * n, n)])

  return k(col3, ew3).reshape(NW, n)


def _edge_pass(g_nodes, row3, col3, ew3, zeros_nf):
  n, f = g_nodes.shape
  npad = zeros_nf.shape[0]
  kch = row3.shape[1]
  rpt = npad // NS

  kp = kch // 2
  pair = 2 * CHUNK
  nbuf = 4
  assert kp >= nbuf and kp % nbuf == 0

  kp0 = (2 * kp * KP0_NUM // (KP0_NUM + KP1_NUM)) // nbuf * nbuf
  kp1 = 2 * kp - kp0
  assert kp1 % nbuf == 0 and kp0 >= nbuf and kp1 >= nbuf
  kpm = max(kp0, kp1)
  flat = lambda a: a.reshape(NW * kp * pair)
  def split(a):
    a = flat(a)
    c0 = a[:NS * kp0 * pair].reshape(NS, kp0, pair)
    c0 = jnp.concatenate(
        [c0, jnp.zeros((NS, kpm - kp0, pair), a.dtype)], axis=1)
    c1 = a[NS * kp0 * pair:].reshape(NS, kp1, pair)
    c1 = jnp.concatenate(
        [c1, jnp.zeros((NS, kpm - kp1, pair), a.dtype)], axis=1)
    return jnp.stack([c0, c1], axis=1).reshape(NW, kpm, pair)
  row4 = split(row3).reshape(NW, kpm, 2, CHUNK)
  col4 = split(col3).reshape(NW, kpm, 2, CHUNK)
  ew2 = split(ew3)
  kp = kpm

  @functools.partial(
      pl.kernel,
      out_type=jax.ShapeDtypeStruct((NC, npad, f), jnp.float32),
      mesh=_sc_mesh(),
      scratch_types=[
          pltpu.VMEM((kp, 2, CHUNK), jnp.int32),
          pltpu.VMEM((kp, 2, CHUNK), jnp.int32),
          pltpu.VMEM((kp, pair), jnp.float32),
          pltpu.VMEM((nbuf, pair, f), jnp.float32),
          pltpu.VMEM_SHARED((npad, f), jnp.float32),
          [pltpu.SemaphoreType.DMA] * nbuf,
          [pltpu.SemaphoreType.DMA] * nbuf,
      ],
      compiler_params=pltpu.CompilerParams(needs_layout_passes=False,
                                           use_tc_tiling_on_sc=False),
  )
  def k(g_hbm, row_hbm, col_hbm, ew_hbm, z_hbm, out_hbm,
        row_v, col_v, ew_v, rbuf, acc_sh, gsem, ssem):
    cid = lax.axis_index("c")
    sid = lax.axis_index("s")
    wid = sid * NC + cid
    base = sid * rpt
    ub = jnp.where(cid == 0, kp0, kp1)

    pltpu.sync_copy(z_hbm.at[pl.ds(base, rpt)], acc_sh.at[pl.ds(base, rpt)])
    pltpu.sync_copy(row_hbm.at[wid], row_v)
    pltpu.sync_copy(col_hbm.at[wid], col_v)
    pltpu.sync_copy(ew_hbm.at[wid], ew_v)
    plsc.subcore_barrier()

    def start_gather(p, b):
      for h in range(2):
        pltpu.async_copy(g_hbm.at[row_v.at[p, h]],
                         rbuf.at[b, pl.ds(h * CHUNK, CHUNK)], gsem[b])

    def wait_gather(b):
      for h in range(2):
        pltpu.make_async_copy(g_hbm.at[row_v.at[0, 0]],
                              rbuf.at[b, pl.ds(h * CHUNK, CHUNK)],
                              gsem[b]).wait()

    def start_scatter(p, b):
      for h in range(2):
        pltpu.async_copy(rbuf.at[b, pl.ds(h * CHUNK, CHUNK)],
                         acc_sh.at[col_v.at[p, h]], ssem[b], add=True)

    def wait_scatter(b):
      for h in range(2):
        pltpu.make_async_copy(rbuf.at[b, pl.ds(h * CHUNK, CHUNK)],
                              acc_sh.at[col_v.at[0, 0]], ssem[b]).wait()

    def scale(p, b):
      @plsc.parallel_loop(0, pair // LANES, unroll=2)
      def _(g):
        wv = ew_v[p, pl.ds(g * LANES, LANES)]
        for l in range(LANES):
          e = g * LANES + l
          w = wv[l]
          for fb in range(f // LANES):
            s = pl.ds(fb * LANES, LANES)
            rbuf[b, e, s] = rbuf[b, e, s] * w

    for b in range(nbuf - 1):
      start_gather(b, b)

    @pl.loop(0, ub, step=nbuf)
    def _(p2):
      for b in range(nbuf):
        p = p2 + b
        prv = (b - 1) % nbuf

        @pl.when(p >= 1)
        def _():
          wait_scatter(prv)

        @pl.when(p + nbuf - 1 < ub)
        def _():
          start_gather(p + nbuf - 1, prv)

        wait_gather(b)
        scale(p, b)
        start_scatter(p, b)

    wait_scatter(3)
    plsc.subcore_barrier()
    pltpu.sync_copy(acc_sh.at[pl.ds(base, rpt)],
                    out_hbm.at[cid, pl.ds(base, rpt)])

  return k(g_nodes, row4, col4, ew2, zeros_nf)


def _tc0(deg_parts):
  nw, n = deg_parts.shape

  def body(deg_ref, dis_ref):
    deg = jnp.sum(deg_ref[...], axis=0) + 1.0
    dis = jnp.where(deg > 0, 1.0 / jnp.sqrt(deg), 0.0)
    dis_ref[...] = dis[:, None]

  return pl.pallas_call(
      body,
      out_shape=jax.ShapeDtypeStruct((n, 1), jnp.float32),
  )(deg_parts)


def _tc1(dis, x, w1, nb):
  n, f_in = x.shape
  hid = w1.shape[1]

  def body(dis_ref, x_ref, w_ref, g_ref):
    g_ref[...] = jnp.dot(x_ref[...], w_ref[...],
                         preferred_element_type=jnp.float32) * dis_ref[...]

  return pl.pallas_call(
      body,
      grid=(n // nb,),
      in_specs=[
          pl.BlockSpec((nb, 1), lambda i: (i, 0)),
          pl.BlockSpec((nb, f_in), lambda i: (i, 0)),
          pl.BlockSpec((f_in, hid), lambda i: (0, 0)),
      ],
      out_specs=pl.BlockSpec((nb, hid), lambda i: (i, 0)),
      out_shape=jax.ShapeDtypeStruct((n, hid), jnp.float32),
  )(dis, x, w1)


def _tc2(acc1, g1, dis, w2, b1, nb):
  n, hid = g1.shape
  c = w2.shape[1]

  def body(acc_ref, g1_ref, dis_ref, w_ref, b_ref, g2_ref):
    a = acc_ref[0] + acc_ref[1] + g1_ref[...]
    out1 = jnp.maximum(a * dis_ref[...] + b_ref[...], 0.0)
    g2_ref[...] = jnp.dot(out1, w_ref[...],
                          preferred_element_type=jnp.float32) * dis_ref[...]

  return pl.pallas_call(
      body,
      grid=(n // nb,),
      in_specs=[
          pl.BlockSpec((NC, nb, hid), lambda i: (0, i, 0)),
          pl.BlockSpec((nb, hid), lambda i: (i, 0)),
          pl.BlockSpec((nb, 1), lambda i: (i, 0)),
          pl.BlockSpec((hid, c), lambda i: (0, 0)),
          pl.BlockSpec((1, hid), lambda i: (0, 0)),
      ],
      out_specs=pl.BlockSpec((nb, c), lambda i: (i, 0)),
      out_shape=jax.ShapeDtypeStruct((n, c), jnp.float32),
  )(acc1, g1, dis, w2, b1)


def _tc3(acc2, g2, dis, b2, nb):
  n, c = g2.shape

  def body(acc_ref, g2_ref, dis_ref, b_ref, o_ref):
    z = (acc_ref[0] + acc_ref[1] + g2_ref[...]) * dis_ref[...] + b_ref[...]
    m = jnp.max(z, axis=1, keepdims=True)
    lse = jnp.log(jnp.sum(jnp.exp(z - m), axis=1, keepdims=True)) + m
    o_ref[...] = z - lse

  return pl.pallas_call(
      body,
      grid=(n // nb,),
      in_specs=[
          pl.BlockSpec((NC, nb, c), lambda i: (0, i, 0)),
          pl.BlockSpec((nb, c), lambda i: (i, 0)),
          pl.BlockSpec((nb, 1), lambda i: (i, 0)),
          pl.BlockSpec((1, c), lambda i: (0, 0)),
      ],
      out_specs=pl.BlockSpec((nb, c), lambda i: (i, 0)),
      out_shape=jax.ShapeDtypeStruct((n, c), jnp.float32),
  )(acc2, g2, dis, b2)


@jax.jit
def kernel(x, edge_index, edge_weight, W1, b1, W2, b2):
  n, _ = x.shape
  hid = W1.shape[1]
  c = W2.shape[1]
  e = edge_weight.shape[0]

  sup = NW * CHUNK
  e_pad = ((e + sup - 1) // sup) * sup
  pad = e_pad - e
  row = jnp.concatenate([edge_index[0], jnp.zeros((pad,), jnp.int32)])
  col = jnp.concatenate([edge_index[1], jnp.zeros((pad,), jnp.int32)])
  ew = jnp.concatenate([edge_weight, jnp.zeros((pad,), jnp.float32)])
  kch = e_pad // sup
  row3 = row.reshape(NW, kch, CHUNK)
  col3 = col.reshape(NW, kch, CHUNK)
  ew3 = ew.reshape(NW, kch, CHUNK)

  nb = 2000
  npad = ((n + 127) // 128) * 128

  deg_parts = _deg_pass(col3, ew3, n)
  dis = _tc0(deg_parts)
  g1 = _tc1(dis, x, W1, nb)
  acc1 = _edge_pass(g1, row3, col3, ew3,
                    jnp.zeros((npad, hid), jnp.float32))
  g2 = _tc2(acc1, g1, dis, W2, b1.reshape(1, hid), nb)
  acc2 = _edge_pass(g2, row3, col3, ew3,
                    jnp.zeros((npad, c), jnp.float32))
  return _tc3(acc2, g2, dis, b2.reshape(1, c), nb)


# speedup vs baseline: 1.0688x; 1.0688x over previous
import functools

import jax
import jax.numpy as jnp
from jax import lax
from jax.experimental import pallas as pl
from jax.experimental.pallas import tpu as pltpu
from jax.experimental.pallas import tpu_sc as plsc

NC = 2
NS = 16
NW = NC * NS
LANES = 16
CHUNK = 128
KP0_NUM, KP1_NUM = 3, 2


def _sc_mesh():
  return plsc.VectorSubcoreMesh(core_axis_name="c", subcore_axis_name="s")


def _deg_pass(col3, ew3, n):
  kch = col3.shape[1]

  @functools.partial(
      pl.kernel,
      out_type=jax.ShapeDtypeStruct((NW * n,), jnp.float32),
      mesh=_sc_mesh(),
      scratch_types=[
          pltpu.VMEM((kch, CHUNK), jnp.int32),
          pltpu.VMEM((kch, CHUNK), jnp.float32),
          pltpu.VMEM((n,), jnp.float32),
      ],
      compiler_params=pltpu.CompilerParams(needs_layout_passes=False),
  )
  def k(col_hbm, ew_hbm, out_hbm, col_v, ew_v, deg_v):
    cid = lax.axis_index("c")
    sid = lax.axis_index("s")
    wid = sid * NC + cid
    pltpu.sync_copy(col_hbm.at[wid], col_v)
    pltpu.sync_copy(ew_hbm.at[wid], ew_v)

    zero = jnp.zeros((LANES,), jnp.float32)

    def zbody(i, carry):
      deg_v[pl.ds(i * LANES, LANES)] = zero
      return carry

    lax.fori_loop(0, n // LANES, zbody, 0)

    def cbody(j, carry):
      for g in range(CHUNK // LANES):
        idx = col_v[j, pl.ds(g * LANES, LANES)]
        val = ew_v[j, pl.ds(g * LANES, LANES)]
        plsc.addupdate_scatter(deg_v, [idx], val)
      return carry

    lax.fori_loop(0, kch, cbody, 0)
    pltpu.sync_copy(deg_v, out_hbm.at[pl.ds(wid * n, n)])

  return k(col3, ew3).reshape(NW, n)


def _edge_pass(g_nodes, row3, col3, ew3, zeros_nf):
  n, f = g_nodes.shape
  npad = zeros_nf.shape[0]
  kch = row3.shape[1]
  rpt = npad // NS

  kp = kch // 2
  pair = 2 * CHUNK
  nbuf = 4
  assert kp >= nbuf and kp % nbuf == 0

  kp0 = (2 * kp * KP0_NUM // (KP0_NUM + KP1_NUM)) // nbuf * nbuf
  kp1 = 2 * kp - kp0
  assert kp1 % nbuf == 0 and kp0 >= nbuf and kp1 >= nbuf
  kpm = max(kp0, kp1)
  flat = lambda a: a.reshape(NW * kp * pair)
  def split(a):
    a = flat(a)
    c0 = a[:NS * kp0 * pair].reshape(NS, kp0, pair)
    c0 = jnp.concatenate(
        [c0, jnp.zeros((NS, kpm - kp0, pair), a.dtype)], axis=1)
    c1 = a[NS * kp0 * pair:].reshape(NS, kp1, pair)
    c1 = jnp.concatenate(
        [c1, jnp.zeros((NS, kpm - kp1, pair), a.dtype)], axis=1)
    return jnp.stack([c0, c1], axis=1).reshape(NW, kpm, pair)
  row4 = split(row3).reshape(NW, kpm, 2, CHUNK)
  col4 = split(col3).reshape(NW, kpm, 2, CHUNK)
  ew2 = split(ew3)
  kp = kpm

  @functools.partial(
      pl.kernel,
      out_type=jax.ShapeDtypeStruct((NC, npad, f), jnp.float32),
      mesh=_sc_mesh(),
      scratch_types=[
          pltpu.VMEM((kp, 2, CHUNK), jnp.int32),
          pltpu.VMEM((kp, 2, CHUNK), jnp.int32),
          pltpu.VMEM((kp, pair), jnp.float32),
          pltpu.VMEM((nbuf, pair, f), jnp.float32),
          pltpu.VMEM_SHARED((npad, f), jnp.float32),
          [pltpu.SemaphoreType.DMA] * nbuf,
          [pltpu.SemaphoreType.DMA] * nbuf,
      ],
      compiler_params=pltpu.CompilerParams(needs_layout_passes=False,
                                           use_tc_tiling_on_sc=False),
  )
  def k(g_hbm, row_hbm, col_hbm, ew_hbm, z_hbm, out_hbm,
        row_v, col_v, ew_v, rbuf, acc_sh, gsem, ssem):
    cid = lax.axis_index("c")
    sid = lax.axis_index("s")
    wid = sid * NC + cid
    base = sid * rpt
    ub = jnp.where(cid == 0, kp0, kp1)

    pltpu.sync_copy(z_hbm.at[pl.ds(base, rpt)], acc_sh.at[pl.ds(base, rpt)])
    pltpu.sync_copy(row_hbm.at[wid], row_v)
    pltpu.sync_copy(col_hbm.at[wid], col_v)
    pltpu.sync_copy(ew_hbm.at[wid], ew_v)
    plsc.subcore_barrier()

    def start_gather(p, b):
      for h in range(2):
        pltpu.async_copy(g_hbm.at[row_v.at[p, h]],
                         rbuf.at[b, pl.ds(h * CHUNK, CHUNK)], gsem[b])

    def wait_gather(b):
      for h in range(2):
        pltpu.make_async_copy(g_hbm.at[row_v.at[0, 0]],
                              rbuf.at[b, pl.ds(h * CHUNK, CHUNK)],
                              gsem[b]).wait()

    def start_scatter(p, b):
      for h in range(2):
        pltpu.async_copy(rbuf.at[b, pl.ds(h * CHUNK, CHUNK)],
                         acc_sh.at[col_v.at[p, h]], ssem[b], add=True)

    def wait_scatter(b):
      for h in range(2):
        pltpu.make_async_copy(rbuf.at[b, pl.ds(h * CHUNK, CHUNK)],
                              acc_sh.at[col_v.at[0, 0]], ssem[b]).wait()

    def scale(p, b):
      @plsc.parallel_loop(0, pair // LANES, unroll=2)
      def _(g):
        wv = ew_v[p, pl.ds(g * LANES, LANES)]
        for l in range(LANES):
          e = g * LANES + l
          w = wv[l]
          for fb in range(f // LANES):
            s = pl.ds(fb * LANES, LANES)
            rbuf[b, e, s] = rbuf[b, e, s] * w

    for b in range(nbuf - 1):
      start_gather(b, b)

    @pl.loop(0, ub, step=nbuf)
    def _(p2):
      for b in range(nbuf):
        p = p2 + b
        prv = (b - 1) % nbuf

        @pl.when(p >= 1)
        def _():
          wait_scatter(prv)

        @pl.when(p + nbuf - 1 < ub)
        def _():
          start_gather(p + nbuf - 1, prv)

        wait_gather(b)
        scale(p, b)
        start_scatter(p, b)

    wait_scatter(3)
    plsc.subcore_barrier()
    pltpu.sync_copy(acc_sh.at[pl.ds(base, rpt)],
                    out_hbm.at[cid, pl.ds(base, rpt)])

  return k(g_nodes, row4, col4, ew2, zeros_nf)


def _tc0(deg_parts):
  nw, n = deg_parts.shape

  def body(deg_ref, dis_ref):
    deg = jnp.sum(deg_ref[...], axis=0) + 1.0
    dis = jnp.where(deg > 0, 1.0 / jnp.sqrt(deg), 0.0)
    dis_ref[...] = dis[:, None]

  return pl.pallas_call(
      body,
      out_shape=jax.ShapeDtypeStruct((n, 1), jnp.float32),
  )(deg_parts)


def _tc1(dis, x, w1, nb):
  n, f_in = x.shape
  hid = w1.shape[1]

  def body(dis_ref, x_ref, w_ref, g_ref):
    g_ref[...] = jnp.dot(x_ref[...], w_ref[...],
                         preferred_element_type=jnp.float32) * dis_ref[...]

  return pl.pallas_call(
      body,
      grid=(n // nb,),
      in_specs=[
          pl.BlockSpec((nb, 1), lambda i: (i, 0)),
          pl.BlockSpec((nb, f_in), lambda i: (i, 0)),
          pl.BlockSpec((f_in, hid), lambda i: (0, 0)),
      ],
      out_specs=pl.BlockSpec((nb, hid), lambda i: (i, 0)),
      out_shape=jax.ShapeDtypeStruct((n, hid), jnp.float32),
  )(dis, x, w1)


def _tc2(acc1, g1, dis, w2, b1, nb):
  n, hid = g1.shape
  c = w2.shape[1]

  def body(acc_ref, g1_ref, dis_ref, w_ref, b_ref, g2_ref):
    a = acc_ref[0] + acc_ref[1] + g1_ref[...]
    out1 = jnp.maximum(a * dis_ref[...] + b_ref[...], 0.0)
    g2_ref[...] = jnp.dot(out1, w_ref[...],
                          preferred_element_type=jnp.float32) * dis_ref[...]

  return pl.pallas_call(
      body,
      grid=(n // nb,),
      in_specs=[
          pl.BlockSpec((NC, nb, hid), lambda i: (0, i, 0)),
          pl.BlockSpec((nb, hid), lambda i: (i, 0)),
          pl.BlockSpec((nb, 1), lambda i: (i, 0)),
          pl.BlockSpec((hid, c), lambda i: (0, 0)),
          pl.BlockSpec((1, hid), lambda i: (0, 0)),
      ],
      out_specs=pl.BlockSpec((nb, c), lambda i: (i, 0)),
      out_shape=jax.ShapeDtypeStruct((n, c), jnp.float32),
  )(acc1, g1, dis, w2, b1)


def _tc3(acc2, g2, dis, b2, nb):
  n, c = g2.shape

  def body(acc_ref, g2_ref, dis_ref, b_ref, o_ref):
    z = (acc_ref[0] + acc_ref[1] + g2_ref[...]) * dis_ref[...] + b_ref[...]
    m = jnp.max(z, axis=1, keepdims=True)
    lse = jnp.log(jnp.sum(jnp.exp(z - m), axis=1, keepdims=True)) + m
    o_ref[...] = z - lse

  return pl.pallas_call(
      body,
      grid=(n // nb,),
      in_specs=[
          pl.BlockSpec((NC, nb, c), lambda i: (0, i, 0)),
          pl.BlockSpec((nb, c), lambda i: (i, 0)),
          pl.BlockSpec((nb, 1), lambda i: (i, 0)),
          pl.BlockSpec((1, c), lambda i: (0, 0)),
      ],
      out_specs=pl.BlockSpec((nb, c), lambda i: (i, 0)),
      out_shape=jax.ShapeDtypeStruct((n, c), jnp.float32),
  )(acc2, g2, dis, b2)


@jax.jit
def kernel(x, edge_index, edge_weight, W1, b1, W2, b2):
  n, _ = x.shape
  hid = W1.shape[1]
  c = W2.shape[1]
  e = edge_weight.shape[0]

  sup = NW * CHUNK
  e_pad = ((e + sup - 1) // sup) * sup
  pad = e_pad - e
  row = jnp.concatenate([edge_index[0], jnp.zeros((pad,), jnp.int32)])
  col = jnp.concatenate([edge_index[1], jnp.zeros((pad,), jnp.int32)])
  ew = jnp.concatenate([edge_weight, jnp.zeros((pad,), jnp.float32)])
  kch = e_pad // sup
  row3 = row.reshape(NW, kch, CHUNK)
  col3 = col.reshape(NW, kch, CHUNK)
  ew3 = ew.reshape(NW, kch, CHUNK)

  nb = 2000
  npad = ((n + 127) // 128) * 128

  deg_parts = _deg_pass(col3, ew3, n)
  dis = _tc0(deg_parts)
  g1 = _tc1(dis, x, W1, nb)
  acc1 = _edge_pass(g1, row3, col3, ew3,
                    jnp.zeros((npad, hid), jnp.float32))
  g2 = _tc2(acc1, g1, dis, W2, b1.reshape(1, hid), nb)
  acc2 = _edge_pass(g2, row3, col3, ew3,
                    jnp.zeros((npad, c), jnp.float32))
  return _tc3(acc2, g2, dis, b2.reshape(1, c), nb)
